# baseline (speedup 1.0000x reference)
import functools

import jax
import jax.numpy as jnp
import numpy as np
from jax.experimental import pallas as pl
from jax.experimental.pallas import tpu as pltpu

B = 2
D_MODEL = 256
N_HEADS = 8
D_FF = 2048
N_LEVELS = 4
N_POINTS = 4
TOPK_SA = 1000
LEVEL_SHAPES = [(64, 64), (32, 32), (16, 16), (8, 8)]
S_PER = sum(h * w for h, w in LEVEL_SHAPES)
TOTAL = B * S_PER
D_HEAD = D_MODEL // N_HEADS


def _layer_norm(x, g, b):
    m = jnp.mean(x, axis=-1, keepdims=True)
    v = jnp.var(x, axis=-1, keepdims=True)
    return (x - m) / jnp.sqrt(v + 1e-5) * g + b


_FFN_BLK = 640


def _ffn_body(x_ref, w1_ref, b1_ref, w2_ref, b2_ref, g_ref, bb_ref, o_ref):
    x = x_ref[...]
    xn = _layer_norm(x, g_ref[...], bb_ref[...])
    h = jnp.dot(xn, w1_ref[...], preferred_element_type=jnp.float32,
                precision=jax.lax.Precision.HIGHEST) + b1_ref[...]
    h = jax.nn.gelu(h)
    o = jnp.dot(h, w2_ref[...], preferred_element_type=jnp.float32,
                precision=jax.lax.Precision.HIGHEST) + b2_ref[...]
    o_ref[...] = x + o + 0.0


def _ffn_pallas(x, W1, b1, W2, b2, g, b):
    n = x.shape[0]
    grid = (n // _FFN_BLK,)
    return pl.pallas_call(
        _ffn_body,
        grid=grid,
        in_specs=[
            pl.BlockSpec((_FFN_BLK, D_MODEL), lambda i: (i, 0)),
            pl.BlockSpec((D_MODEL, D_FF), lambda i: (0, 0)),
            pl.BlockSpec((D_FF,), lambda i: (0,)),
            pl.BlockSpec((D_FF, D_MODEL), lambda i: (0, 0)),
            pl.BlockSpec((D_MODEL,), lambda i: (0,)),
            pl.BlockSpec((D_MODEL,), lambda i: (0,)),
            pl.BlockSpec((D_MODEL,), lambda i: (0,)),
        ],
        out_specs=pl.BlockSpec((_FFN_BLK, D_MODEL), lambda i: (i, 0)),
        out_shape=jax.ShapeDtypeStruct((n, D_MODEL), jnp.float32),
    )(x, W1, b1, W2, b2, g, b)


def _self_attn_block(x, pos, Wq, Wk, Wv, Wo, g, b):
    Bq, L, d = x.shape
    x2 = _layer_norm(x, g, b)
    q = (x2 + pos) @ Wq
    k = (x2 + pos) @ Wk
    v = x2 @ Wv
    q = q.reshape(Bq, L, N_HEADS, D_HEAD).transpose(0, 2, 1, 3)
    k = k.reshape(Bq, L, N_HEADS, D_HEAD).transpose(0, 2, 1, 3)
    v = v.reshape(Bq, L, N_HEADS, D_HEAD).transpose(0, 2, 1, 3)
    logits = jnp.einsum("bhqd,bhkd->bhqk", q, k) / np.sqrt(D_HEAD)
    a = jax.nn.softmax(logits, axis=-1)
    o = jnp.einsum("bhqk,bhkd->bhqd", a, v).transpose(0, 2, 1, 3).reshape(Bq, L, d) @ Wo
    return x + o


def _gather_heads(v_flat, lin):
    S, Hh, P = lin.shape
    vp = v_flat.transpose(1, 0, 2)
    lp = lin.transpose(1, 0, 2).reshape(Hh, S * P)
    g = jnp.take_along_axis(vp, lp[:, :, None], axis=1)
    return g.reshape(Hh, S, P, -1)


def _msdeform_block(x_flat, pos_flat, ref_xy, value, spatial_shapes, Wv, bv, Woff, boff, Waw, baw, Wout, bout, g, b):
    shapes = np.asarray(LEVEL_SHAPES)
    starts = np.concatenate([[0], np.cumsum(shapes[:, 0] * shapes[:, 1])])
    x = x_flat.reshape(B, S_PER, D_MODEL)
    pos = pos_flat.reshape(B, S_PER, D_MODEL)
    ref = ref_xy.reshape(B, S_PER, 2)
    x2 = _layer_norm(x, g, b)
    q = x2 + pos
    v = value @ Wv + bv
    off = (q @ Woff + boff).reshape(B, S_PER, N_HEADS, N_LEVELS, N_POINTS, 2)
    aw = jax.nn.softmax((q @ Waw + baw).reshape(B, S_PER, N_HEADS, N_LEVELS * N_POINTS), axis=-1)
    aw = aw.reshape(B, S_PER, N_HEADS, N_LEVELS, N_POINTS)
    normalizer = spatial_shapes[:, ::-1].astype(jnp.float32)
    loc = ref[:, :, None, None, None, :] + off / normalizer[None, None, None, :, None, :]
    out = jnp.zeros((B, N_HEADS, S_PER, D_HEAD), jnp.float32)
    for l in range(N_LEVELS):
        Hl, Wl = int(shapes[l, 0]), int(shapes[l, 1])
        v_l = v[:, int(starts[l]):int(starts[l + 1])].reshape(B, Hl * Wl, N_HEADS, D_HEAD)
        loc_l = loc[:, :, :, l, :, :]
        xpix = loc_l[..., 0] * Wl - 0.5
        ypix = loc_l[..., 1] * Hl - 0.5
        x0 = jnp.floor(xpix)
        y0 = jnp.floor(ypix)
        wx1 = xpix - x0
        wy1 = ypix - y0
        x0i = x0.astype(jnp.int32)
        y0i = y0.astype(jnp.int32)
        acc = 0.0
        for dx, dy in ((0, 0), (1, 0), (0, 1), (1, 1)):
            xi = x0i + dx
            yi = y0i + dy
            wgt = (wx1 if dx else (1.0 - wx1)) * (wy1 if dy else (1.0 - wy1))
            valid = ((xi >= 0) & (xi < Wl) & (yi >= 0) & (yi < Hl)).astype(jnp.float32)
            lin = jnp.clip(yi, 0, Hl - 1) * Wl + jnp.clip(xi, 0, Wl - 1)
            gsamp = jax.vmap(_gather_heads)(v_l, lin)
            acc = acc + gsamp * (wgt * valid).transpose(0, 2, 1, 3)[..., None]
        out = out + (acc * aw[:, :, :, l, :].transpose(0, 2, 1, 3)[..., None]).sum(axis=3)
    out = out.transpose(0, 2, 1, 3).reshape(B, S_PER, D_MODEL) @ Wout + bout
    return (x + out).reshape(TOTAL, D_MODEL)


def kernel(queries, query_pos_encoding, query_ij_indices, query_normalized_xy_positions, batch_offsets, stacked_feature_maps, spatial_shapes, token_predicted_salience, token_electron_probs, sa_Wq, sa_Wk, sa_Wv, sa_Wo, sa_ln_g, sa_ln_b, da_value_W, da_value_b, da_off_W, da_off_b, da_attn_W, da_attn_b, da_out_W, da_out_b, da_ln_g, da_ln_b, ffn_W1, ffn_b1, ffn_W2, ffn_b2, ffn_ln_g, ffn_ln_b):
    token_scores = token_electron_probs * jax.nn.sigmoid(token_predicted_salience)
    scores_b = token_scores.reshape(B, S_PER)
    queries_b = queries.reshape(B, S_PER, D_MODEL)
    pos_b = query_pos_encoding.reshape(B, S_PER, D_MODEL)
    _, indices = jax.lax.top_k(scores_b, TOPK_SA)
    idx3 = jnp.broadcast_to(indices[:, :, None], (B, TOPK_SA, D_MODEL))
    sel_q = jnp.take_along_axis(queries_b, idx3, axis=1)
    sel_pos = jnp.take_along_axis(pos_b, idx3, axis=1)
    sa_out = _self_attn_block(sel_q, sel_pos, sa_Wq, sa_Wk, sa_Wv, sa_Wo, sa_ln_g, sa_ln_b)
    bidx = jnp.arange(B)[:, None]
    queries_b = queries_b.at[bidx, indices].set(sa_out)
    queries_2 = queries_b.reshape(TOTAL, D_MODEL)
    queries_3 = _msdeform_block(queries_2, query_pos_encoding, query_normalized_xy_positions, stacked_feature_maps, spatial_shapes, da_value_W, da_value_b, da_off_W, da_off_b, da_attn_W, da_attn_b, da_out_W, da_out_b, da_ln_g, da_ln_b)
    queries_4 = _ffn_pallas(queries_3, ffn_W1, ffn_b1, ffn_W2, ffn_b2, ffn_ln_g, ffn_ln_b)
    return queries_4


# speedup vs baseline: 39.3657x; 39.3657x over previous
import functools

import jax
import jax.numpy as jnp
import numpy as np
from jax import lax
from jax.experimental import pallas as pl
from jax.experimental.pallas import tpu as pltpu
from jax.experimental.pallas import tpu_sc as plsc

B = 2
D_MODEL = 256
N_HEADS = 8
D_FF = 2048
N_LEVELS = 4
N_POINTS = 4
TOPK_SA = 1000
LEVEL_SHAPES = [(64, 64), (32, 32), (16, 16), (8, 8)]
S_PER = sum(h * w for h, w in LEVEL_SHAPES)
TOTAL = B * S_PER
D_HEAD = D_MODEL // N_HEADS

_BLK = 640
_NW = 32
_TPW = TOTAL // _NW


def _layer_norm(x, g, b):
    m = jnp.mean(x, axis=-1, keepdims=True)
    v = jnp.var(x, axis=-1, keepdims=True)
    return (x - m) / jnp.sqrt(v + 1e-5) * g + b


def _hp_dot(a, b):
    return jnp.dot(a, b, preferred_element_type=jnp.float32,
                   precision=jax.lax.Precision.HIGHEST)


def _proj_body(x_ref, pos_ref, ref_ref, feat_ref, wv_ref, bv_ref, woff_ref,
               boff_ref, waw_ref, baw_ref, g_ref, b_ref, v_ref, r_ref, w_ref):
    i = pl.program_id(0)
    x = x_ref[...]
    xn = _layer_norm(x, g_ref[...], b_ref[...])
    q = xn + pos_ref[...]

    v_ref[...] = _hp_dot(feat_ref[...], wv_ref[...]) + bv_ref[...]

    off2 = _hp_dot(q, woff_ref[...]) + boff_ref[...]
    offx = off2[:, :128]
    offy = off2[:, 128:]

    att = _hp_dot(q, waw_ref[...]) + baw_ref[...]
    aw = jax.nn.softmax(att.reshape(_BLK, N_HEADS, 16), axis=-1)
    aw = aw.reshape(_BLK, 128)

    lane = lax.broadcasted_iota(jnp.int32, (_BLK, 128), 1)
    lvl = (lane // 4) % 4
    h_of_lane = lane // 16
    wl_i = 64 >> lvl
    wl_f = wl_i.astype(jnp.float32)
    start = jnp.where(lvl == 0, 0,
                      jnp.where(lvl == 1, 4096,
                                jnp.where(lvl == 2, 5120, 5376)))

    t_idx = i * _BLK + lax.broadcasted_iota(jnp.int32, (_BLK, 128), 0)
    b_idx = t_idx // S_PER

    refxy = ref_ref[...]
    refx = jnp.broadcast_to(refxy[:, 0:1], (_BLK, 128))
    refy = jnp.broadcast_to(refxy[:, 1:2], (_BLK, 128))

    xpix = (refx + offx / wl_f) * wl_f - 0.5
    ypix = (refy + offy / wl_f) * wl_f - 0.5
    x0 = jnp.floor(xpix)
    y0 = jnp.floor(ypix)
    wx1 = xpix - x0
    wy1 = ypix - y0
    x0i = x0.astype(jnp.int32)
    y0i = y0.astype(jnp.int32)

    qx = jnp.clip(x0i, 0, wl_i - 1)
    qy = jnp.clip(y0i, 0, wl_i - 1)
    r_ref[...] = (b_idx * S_PER + start + qy * wl_i + qx) * N_HEADS + h_of_lane

    f1 = jnp.float32(1.0)
    wxa = (f1 - wx1) * (x0i == qx) + wx1 * (x0i + 1 == qx)
    wxb = ((f1 - wx1) * (x0i == qx + 1) + wx1 * (x0i + 1 == qx + 1)) \
        * (qx + 1 <= wl_i - 1)
    wya = (f1 - wy1) * (y0i == qy) + wy1 * (y0i + 1 == qy)
    wyb = ((f1 - wy1) * (y0i == qy + 1) + wy1 * (y0i + 1 == qy + 1)) \
        * (qy + 1 <= wl_i - 1)
    w_ref[...] = jnp.concatenate(
        [wya * wxa * aw, wya * wxb * aw, wyb * wxa * aw, wyb * wxb * aw],
        axis=1)


def _proj_pallas(q2, pos, refxy, feat, Wv, bv, Woff_p, boff_p, Waw, baw, g, b):
    grid = (TOTAL // _BLK,)
    return pl.pallas_call(
        _proj_body,
        grid=grid,
        in_specs=[
            pl.BlockSpec((_BLK, D_MODEL), lambda i: (i, 0)),
            pl.BlockSpec((_BLK, D_MODEL), lambda i: (i, 0)),
            pl.BlockSpec((_BLK, 2), lambda i: (i, 0)),
            pl.BlockSpec((_BLK, D_MODEL), lambda i: (i, 0)),
            pl.BlockSpec((D_MODEL, D_MODEL), lambda i: (0, 0)),
            pl.BlockSpec((D_MODEL,), lambda i: (0,)),
            pl.BlockSpec((D_MODEL, D_MODEL), lambda i: (0, 0)),
            pl.BlockSpec((D_MODEL,), lambda i: (0,)),
            pl.BlockSpec((D_MODEL, 128), lambda i: (0, 0)),
            pl.BlockSpec((128,), lambda i: (0,)),
            pl.BlockSpec((D_MODEL,), lambda i: (0,)),
            pl.BlockSpec((D_MODEL,), lambda i: (0,)),
        ],
        out_specs=[
            pl.BlockSpec((_BLK, D_MODEL), lambda i: (i, 0)),
            pl.BlockSpec((_BLK, 128), lambda i: (i, 0)),
            pl.BlockSpec((_BLK, 512), lambda i: (i, 0)),
        ],
        out_shape=[
            jax.ShapeDtypeStruct((TOTAL, D_MODEL), jnp.float32),
            jax.ShapeDtypeStruct((TOTAL, 128), jnp.int32),
            jax.ShapeDtypeStruct((TOTAL, 512), jnp.float32),
        ],
    )(q2, pos, refxy, feat, Wv, bv, Woff_p, boff_p, Waw, baw, g, b)


def _build_quads(v):
    v3 = v.reshape(B, S_PER, N_HEADS, D_HEAD)
    parts = []
    off = 0
    for (H, W) in LEVEL_SHAPES:
        Vl = v3[:, off:off + H * W].reshape(B, H, W, N_HEADS, D_HEAD)
        Sx = jnp.concatenate([Vl[:, :, 1:], Vl[:, :, -1:]], axis=2)
        Sy = jnp.concatenate([Vl[:, 1:], Vl[:, -1:]], axis=1)
        Sxy = jnp.concatenate([Sx[:, 1:], Sx[:, -1:]], axis=1)
        Ql = jnp.stack([Vl, Sx, Sy, Sxy], axis=4)
        parts.append(Ql.reshape(B, H * W, N_HEADS, 4 * D_HEAD))
        off += H * W
    return jnp.concatenate(parts, axis=1).reshape(TOTAL * N_HEADS, 4 * D_HEAD)


def _sample_sc(quads, R, W):
    mesh = plsc.VectorSubcoreMesh(core_axis_name="c", subcore_axis_name="s")
    cp = pltpu.CompilerParams(needs_layout_passes=False)

    @functools.partial(
        pl.kernel,
        mesh=mesh,
        compiler_params=cp,
        out_type=jax.ShapeDtypeStruct((TOTAL, D_MODEL), jnp.float32),
        scratch_types=[
            pltpu.VMEM((128,), jnp.int32),
            pltpu.VMEM((512,), jnp.float32),
            pltpu.VMEM((128, 128), jnp.float32),
            pltpu.VMEM((256,), jnp.float32),
            pltpu.SemaphoreType.DMA,
        ],
    )
    def k(q_hbm, r_hbm, w_hbm, o_hbm, idx_v, w_v, g_v, out_v, sem):
        wid = lax.axis_index("s") * 2 + lax.axis_index("c")
        base = wid * _TPW

        @pl.loop(0, _TPW)
        def _(i):
            t = base + i
            pltpu.sync_copy(r_hbm.at[t], idx_v)
            pltpu.sync_copy(w_hbm.at[t], w_v)
            pltpu.async_copy(q_hbm.at[idx_v], g_v, sem).wait()

            zero16 = jnp.zeros((16,), jnp.float32)
            for h in range(N_HEADS):
                def body(lane, accs, h=h):
                    row = h * 16 + lane
                    row16 = jnp.full((16,), row, jnp.int32)
                    new = []
                    for slot in range(4):
                        w16 = plsc.load_gather(w_v, [row16 + slot * 128])
                        g0 = g_v[row, pl.ds(slot * 32, 16)]
                        g1 = g_v[row, pl.ds(slot * 32 + 16, 16)]
                        new.append(accs[2 * slot] + w16 * g0)
                        new.append(accs[2 * slot + 1] + w16 * g1)
                    return tuple(new)

                accs = lax.fori_loop(0, 16, body,
                                     tuple(zero16 for _ in range(8)))
                out_v[pl.ds(h * 32, 16)] = \
                    (accs[0] + accs[2]) + (accs[4] + accs[6])
                out_v[pl.ds(h * 32 + 16, 16)] = \
                    (accs[1] + accs[3]) + (accs[5] + accs[7])
            pltpu.sync_copy(out_v, o_hbm.at[t])

    return k(quads, R, W)


def _outffn_body(x_ref, s_ref, wo_ref, bo_ref, w1_ref, b1_ref, w2_ref, b2_ref,
                 g_ref, bb_ref, o_ref):
    x3 = x_ref[...] + _hp_dot(s_ref[...], wo_ref[...]) + bo_ref[...]
    xn = _layer_norm(x3, g_ref[...], bb_ref[...])
    h = jax.nn.gelu(_hp_dot(xn, w1_ref[...]) + b1_ref[...])
    o_ref[...] = x3 + _hp_dot(h, w2_ref[...]) + b2_ref[...]


def _outffn_pallas(x, s, Wo, bo, W1, b1, W2, b2, g, b):
    grid = (TOTAL // _BLK,)
    return pl.pallas_call(
        _outffn_body,
        grid=grid,
        in_specs=[
            pl.BlockSpec((_BLK, D_MODEL), lambda i: (i, 0)),
            pl.BlockSpec((_BLK, D_MODEL), lambda i: (i, 0)),
            pl.BlockSpec((D_MODEL, D_MODEL), lambda i: (0, 0)),
            pl.BlockSpec((D_MODEL,), lambda i: (0,)),
            pl.BlockSpec((D_MODEL, D_FF), lambda i: (0, 0)),
            pl.BlockSpec((D_FF,), lambda i: (0,)),
            pl.BlockSpec((D_FF, D_MODEL), lambda i: (0, 0)),
            pl.BlockSpec((D_MODEL,), lambda i: (0,)),
            pl.BlockSpec((D_MODEL,), lambda i: (0,)),
            pl.BlockSpec((D_MODEL,), lambda i: (0,)),
        ],
        out_specs=pl.BlockSpec((_BLK, D_MODEL), lambda i: (i, 0)),
        out_shape=jax.ShapeDtypeStruct((TOTAL, D_MODEL), jnp.float32),
    )(x, s, Wo, bo, W1, b1, W2, b2, g, b)


def _sa_body(x_ref, pos_ref, wq_ref, wk_ref, wv_ref, wo_ref, g_ref, b_ref,
             o_ref):
    x = x_ref[0]
    x2 = _layer_norm(x, g_ref[...], b_ref[...])
    qin = x2 + pos_ref[0]
    q = _hp_dot(qin, wq_ref[...])
    k = _hp_dot(qin, wk_ref[...])
    v = _hp_dot(x2, wv_ref[...])
    scale = jnp.float32(1.0 / np.sqrt(D_HEAD))
    outs = []
    for h in range(N_HEADS):
        qh = q[:, h * D_HEAD:(h + 1) * D_HEAD] * scale
        kh = k[:, h * D_HEAD:(h + 1) * D_HEAD]
        vh = v[:, h * D_HEAD:(h + 1) * D_HEAD]
        logits = jnp.dot(qh, kh.T, preferred_element_type=jnp.float32,
                         precision=jax.lax.Precision.HIGHEST)
        a = jax.nn.softmax(logits, axis=-1)
        outs.append(jnp.dot(a, vh, preferred_element_type=jnp.float32,
                            precision=jax.lax.Precision.HIGHEST))
    o = jnp.concatenate(outs, axis=1)
    o_ref[0] = x + _hp_dot(o, wo_ref[...])


def _sa_pallas(x, pos, Wq, Wk, Wv, Wo, g, b):
    return pl.pallas_call(
        _sa_body,
        grid=(B,),
        in_specs=[
            pl.BlockSpec((1, TOPK_SA, D_MODEL), lambda i: (i, 0, 0)),
            pl.BlockSpec((1, TOPK_SA, D_MODEL), lambda i: (i, 0, 0)),
            pl.BlockSpec((D_MODEL, D_MODEL), lambda i: (0, 0)),
            pl.BlockSpec((D_MODEL, D_MODEL), lambda i: (0, 0)),
            pl.BlockSpec((D_MODEL, D_MODEL), lambda i: (0, 0)),
            pl.BlockSpec((D_MODEL, D_MODEL), lambda i: (0, 0)),
            pl.BlockSpec((D_MODEL,), lambda i: (0,)),
            pl.BlockSpec((D_MODEL,), lambda i: (0,)),
        ],
        out_specs=pl.BlockSpec((1, TOPK_SA, D_MODEL), lambda i: (i, 0, 0)),
        out_shape=jax.ShapeDtypeStruct((B, TOPK_SA, D_MODEL), jnp.float32),
    )(x, pos, Wq, Wk, Wv, Wo, g, b)


def _offset_perm():
    perm = np.zeros(256, np.int32)
    for h in range(8):
        for l in range(4):
            for p in range(4):
                for xy in range(2):
                    perm[xy * 128 + h * 16 + l * 4 + p] = h * 32 + l * 8 + p * 2 + xy
    return perm


_OFF_PERM = _offset_perm()


def kernel(queries, query_pos_encoding, query_ij_indices, query_normalized_xy_positions, batch_offsets, stacked_feature_maps, spatial_shapes, token_predicted_salience, token_electron_probs, sa_Wq, sa_Wk, sa_Wv, sa_Wo, sa_ln_g, sa_ln_b, da_value_W, da_value_b, da_off_W, da_off_b, da_attn_W, da_attn_b, da_out_W, da_out_b, da_ln_g, da_ln_b, ffn_W1, ffn_b1, ffn_W2, ffn_b2, ffn_ln_g, ffn_ln_b):
    token_scores = token_electron_probs * jax.nn.sigmoid(token_predicted_salience)
    scores_b = token_scores.reshape(B, S_PER)
    queries_b = queries.reshape(B, S_PER, D_MODEL)
    pos_b = query_pos_encoding.reshape(B, S_PER, D_MODEL)
    _, indices = jax.lax.top_k(scores_b, TOPK_SA)
    idx3 = jnp.broadcast_to(indices[:, :, None], (B, TOPK_SA, D_MODEL))
    sel_q = jnp.take_along_axis(queries_b, idx3, axis=1)
    sel_pos = jnp.take_along_axis(pos_b, idx3, axis=1)
    sa_out = _sa_pallas(sel_q, sel_pos, sa_Wq, sa_Wk, sa_Wv, sa_Wo,
                        sa_ln_g, sa_ln_b)
    bidx = jnp.arange(B)[:, None]
    queries_b = queries_b.at[bidx, indices].set(sa_out)
    queries_2 = queries_b.reshape(TOTAL, D_MODEL)

    woff_p = da_off_W[:, _OFF_PERM]
    boff_p = da_off_b[_OFF_PERM]
    feat = stacked_feature_maps.reshape(TOTAL, D_MODEL)
    v, R, W = _proj_pallas(queries_2, query_pos_encoding,
                           query_normalized_xy_positions, feat,
                           da_value_W, da_value_b, woff_p, boff_p,
                           da_attn_W, da_attn_b, da_ln_g, da_ln_b)
    sampled = _sample_sc(_build_quads(v), R, W)
    return _outffn_pallas(queries_2, sampled, da_out_W, da_out_b,
                          ffn_W1, ffn_b1, ffn_W2, ffn_b2, ffn_ln_g, ffn_ln_b)
